# Initial kernel scaffold; baseline (speedup 1.0000x reference)
import jax
import jax.numpy as jnp
from jax.experimental import pallas as pl


def kernel(x, edge_index, W1, R1, b1, g1, be1, W2, R2, b2, g2, be2, W3, R3, b3):
    raise NotImplementedError("write your pallas kernel here")


# speedup vs baseline: 2.2472x; 2.2472x over previous
import functools

import jax
import jax.numpy as jnp
from jax import lax
from jax.experimental import pallas as pl
from jax.experimental.pallas import tpu as pltpu
from jax.experimental.pallas import tpu_sc as plsc

N = 10000
NPAD = 10240
NW = 32
CHUNK = 128
CPW = 80
GRP = 8
NGRP = CPW // GRP
EPAD = NW * CPW * CHUNK
RPT = NPAD // 16


def _make_agg(d_feat, with_deg):
  mesh = plsc.VectorSubcoreMesh(core_axis_name="c", subcore_axis_name="s")
  out_type = [jax.ShapeDtypeStruct((2, NPAD, d_feat), jnp.float32)]
  scratch = [
      pltpu.VMEM((GRP, CHUNK), jnp.int32),
      pltpu.VMEM((GRP, CHUNK), jnp.int32),
      pltpu.VMEM((CHUNK, d_feat), jnp.float32),
      pltpu.VMEM_SHARED((NPAD, d_feat), jnp.float32),
      pltpu.SemaphoreType.DMA,
  ]
  del with_deg
  if False:
    pass
  if True:
    def body(table, row2d, col2d,
             out, idx_r, idx_c, gbuf, acc, sem):
      c = lax.axis_index("c")
      s = lax.axis_index("s")
      g = c * 16 + s
      r0 = s * RPT

      def zero_gbuf(j, carry):
        for l in range(d_feat // 16):
          gbuf[j, pl.ds(l * 16, 16)] = jnp.zeros((16,), jnp.float32)
        return carry
      lax.fori_loop(0, CHUNK, zero_gbuf, 0)
      for t in range(RPT // CHUNK):
        pltpu.sync_copy(gbuf, acc.at[pl.ds(r0 + t * CHUNK, CHUNK)])
      plsc.subcore_barrier()

      def grp(k, carry):
        base = g * CPW + k * GRP
        pltpu.sync_copy(row2d.at[pl.ds(base, GRP)], idx_r)
        pltpu.sync_copy(col2d.at[pl.ds(base, GRP)], idx_c)
        for j in range(GRP):
          pltpu.async_copy(table.at[idx_c.at[j]], gbuf, sem).wait()
          pltpu.sync_copy(gbuf, acc.at[idx_r.at[j]], add=True)
        return carry

      lax.fori_loop(0, NGRP, grp, 0)
      plsc.subcore_barrier()
      pltpu.sync_copy(acc.at[pl.ds(r0, RPT)], out.at[c, pl.ds(r0, RPT)])

  return pl.kernel(body, out_type=out_type, mesh=mesh, scratch_types=scratch)


BLK = 400
NBLK = N // BLK


def _dense_bn_body(s_ref, deg_ref, h_ref, w_ref, r_ref, b_ref, g_ref, be_ref,
                   o_ref, u_ref, stat_ref):
  p = pl.program_id(0)
  i = pl.program_id(1)

  @pl.when(p == 0)
  def _compute():
    @pl.when(i == 0)
    def _init():
      stat_ref[...] = jnp.zeros_like(stat_ref)

    ssum = s_ref[0] + s_ref[1]
    deg = jnp.maximum(deg_ref[0] + deg_ref[1], 1.0)
    agg = ssum / deg
    u = (jnp.dot(agg, w_ref[...], preferred_element_type=jnp.float32)
         + jnp.dot(h_ref[...], r_ref[...], preferred_element_type=jnp.float32)
         + b_ref[...])
    u_ref[pl.ds(i * BLK, BLK), :] = u
    stat_ref[0:1, :] += jnp.sum(u, axis=0, keepdims=True)
    stat_ref[1:2, :] += jnp.sum(u * u, axis=0, keepdims=True)
    o_ref[...] = u

  @pl.when(p == 1)
  def _normalize():
    mu = stat_ref[0:1, :] * (1.0 / N)
    var = stat_ref[1:2, :] * (1.0 / N) - mu * mu
    inv = lax.rsqrt(var + 1e-5)
    u = u_ref[pl.ds(i * BLK, BLK), :]
    o_ref[...] = jnp.maximum((u - mu) * inv * g_ref[...] + be_ref[...], 0.0)


def _dense_bn(s2, deg, h, W, R, b, g, be):
  return pl.pallas_call(
      _dense_bn_body,
      grid=(2, NBLK),
      in_specs=[
          pl.BlockSpec((2, BLK, 128), lambda p, i: (0, i, 0)),
          pl.BlockSpec((2, BLK, 1), lambda p, i: (0, i, 0)),
          pl.BlockSpec((BLK, 128), lambda p, i: (i, 0)),
          pl.BlockSpec((128, 128), lambda p, i: (0, 0)),
          pl.BlockSpec((128, 128), lambda p, i: (0, 0)),
          pl.BlockSpec((1, 128), lambda p, i: (0, 0)),
          pl.BlockSpec((1, 128), lambda p, i: (0, 0)),
          pl.BlockSpec((1, 128), lambda p, i: (0, 0)),
      ],
      out_specs=pl.BlockSpec((BLK, 128), lambda p, i: (i, 0)),
      out_shape=jax.ShapeDtypeStruct((N, 128), jnp.float32),
      scratch_shapes=[
          pltpu.VMEM((N, 128), jnp.float32),
          pltpu.VMEM((2, 128), jnp.float32),
      ],
  )(s2, deg, h, W, R, b, g, be)


def _final_body(s_ref, deg_ref, h_ref, w3_ref, r3_ref, b3_ref, o_ref):
  ssum = s_ref[0] + s_ref[1]
  deg = jnp.maximum(deg_ref[0] + deg_ref[1], 1.0)
  agg = ssum / deg
  u = (jnp.dot(agg, w3_ref[...], preferred_element_type=jnp.float32)
       + jnp.dot(h_ref[...], r3_ref[...], preferred_element_type=jnp.float32)
       + b3_ref[...])
  lane = lax.broadcasted_iota(jnp.int32, u.shape, 1)
  um = jnp.where(lane < 40, u, -1e30)
  m = jnp.max(um, axis=1, keepdims=True)
  lse = jnp.log(jnp.sum(jnp.exp(um - m), axis=1, keepdims=True)) + m
  o_ref[...] = (um - lse)[:, :40]


def _final(s3, deg, h2, W3p, R3p, b3p):
  return pl.pallas_call(
      _final_body,
      grid=(NBLK,),
      in_specs=[
          pl.BlockSpec((2, BLK, 128), lambda i: (0, i, 0)),
          pl.BlockSpec((2, BLK, 1), lambda i: (0, i, 0)),
          pl.BlockSpec((BLK, 128), lambda i: (i, 0)),
          pl.BlockSpec((128, 64), lambda i: (0, 0)),
          pl.BlockSpec((128, 64), lambda i: (0, 0)),
          pl.BlockSpec((1, 64), lambda i: (0, 0)),
      ],
      out_specs=pl.BlockSpec((BLK, 40), lambda i: (i, 0)),
      out_shape=jax.ShapeDtypeStruct((N, 40), jnp.float32),
  )(s3, deg, h2, W3p, R3p, b3p)


_agg128 = _make_agg(128, False)


def kernel(x, edge_index, W1, R1, b1, g1, be1, W2, R2, b2, g2, be2, W3, R3, b3):
  E = edge_index.shape[1]
  row = edge_index[0]
  col = edge_index[1]
  row2d = jnp.concatenate(
      [row, jnp.full((EPAD - E,), N, jnp.int32)]).reshape(NW * CPW, CHUNK)
  col2d = jnp.concatenate(
      [col, jnp.zeros((EPAD - E,), jnp.int32)]).reshape(NW * CPW, CHUNK)
  b1r = b1.reshape(1, 128)
  g1r = g1.reshape(1, 128)
  be1r = be1.reshape(1, 128)
  b2r = b2.reshape(1, 128)
  g2r = g2.reshape(1, 128)
  be2r = be2.reshape(1, 128)
  W3p = jnp.pad(W3, ((0, 0), (0, 24)))
  R3p = jnp.pad(R3, ((0, 0), (0, 24)))
  b3p = jnp.pad(b3, (0, 24)).reshape(1, 64)

  dg = jax.ops.segment_sum(jnp.ones((E,), jnp.float32), row, num_segments=NPAD)
  deg = jnp.stack([dg[:, None], jnp.zeros((NPAD, 1), jnp.float32)])
  (s1,) = _agg128(x, row2d, col2d)
  h1 = _dense_bn(s1, deg, x, W1, R1, b1r, g1r, be1r)
  (s2,) = _agg128(h1, row2d, col2d)
  h2 = _dense_bn(s2, deg, h1, W2, R2, b2r, g2r, be2r)
  (s3,) = _agg128(h2, row2d, col2d)
  return _final(s3, deg, h2, W3p, R3p, b3p)


# speedup vs baseline: 2.4360x; 1.0840x over previous
import functools

import jax
import jax.numpy as jnp
from jax import lax
from jax.experimental import pallas as pl
from jax.experimental.pallas import tpu as pltpu
from jax.experimental.pallas import tpu_sc as plsc

N = 10000
NPAD = 10240
NW = 32
CHUNK = 128
CPW = 80
GRP = 8
NGRP = CPW // GRP
EPAD = NW * CPW * CHUNK
RPT = NPAD // 16


def _make_agg(d_feat, with_deg):
  mesh = plsc.VectorSubcoreMesh(core_axis_name="c", subcore_axis_name="s")
  out_type = [jax.ShapeDtypeStruct((2, NPAD, d_feat), jnp.float32)]
  scratch = [
      pltpu.VMEM((2, GRP, CHUNK), jnp.int32),
      pltpu.VMEM((2, GRP, CHUNK), jnp.int32),
      pltpu.VMEM((CHUNK, d_feat), jnp.float32),
      pltpu.VMEM((CHUNK, d_feat), jnp.float32),
      pltpu.VMEM_SHARED((NPAD, d_feat), jnp.float32),
      pltpu.SemaphoreType.DMA,
      pltpu.SemaphoreType.DMA,
      pltpu.SemaphoreType.DMA,
  ]
  del with_deg

  if True:
    def body(table, row2d, col2d,
             out, idx_r, idx_c, g0, g1, acc, sem0, sem1, isem):
      c = lax.axis_index("c")
      s = lax.axis_index("s")
      g = c * 16 + s
      r0 = s * RPT

      def zero_gbuf(j, carry):
        for l in range(d_feat // 16):
          g0[j, pl.ds(l * 16, 16)] = jnp.zeros((16,), jnp.float32)
        return carry
      lax.fori_loop(0, CHUNK, zero_gbuf, 0)
      for t in range(RPT // CHUNK):
        pltpu.sync_copy(g0, acc.at[pl.ds(r0 + t * CHUNK, CHUNK)])
      pltpu.sync_copy(row2d.at[pl.ds(g * CPW, GRP)], idx_r.at[0])
      pltpu.sync_copy(col2d.at[pl.ds(g * CPW, GRP)], idx_c.at[0])
      plsc.subcore_barrier()

      bufs = [g0, g1]
      sems = [sem0, sem1]

      def grp(k, carry):
        cur = lax.rem(k, 2)
        nxt = lax.rem(k + 1, 2)
        nbase = g * CPW + (k + 1) * GRP

        @pl.when(k < NGRP - 1)
        def _prefetch_idx():
          pltpu.async_copy(row2d.at[pl.ds(nbase, GRP)], idx_r.at[nxt], isem)
          pltpu.async_copy(col2d.at[pl.ds(nbase, GRP)], idx_c.at[nxt], isem)

        descs = [None] * GRP
        descs[0] = pltpu.async_copy(table.at[idx_c.at[cur, 0]], bufs[0],
                                    sems[0])
        for j in range(GRP):
          if j + 1 < GRP:
            descs[j + 1] = pltpu.async_copy(
                table.at[idx_c.at[cur, j + 1]], bufs[(j + 1) % 2],
                sems[(j + 1) % 2])
          descs[j].wait()
          pltpu.sync_copy(bufs[j % 2], acc.at[idx_r.at[cur, j]], add=True)

        @pl.when(k < NGRP - 1)
        def _wait_idx():
          pltpu.make_async_copy(row2d.at[pl.ds(nbase, GRP)], idx_r.at[nxt],
                                isem).wait()
          pltpu.make_async_copy(col2d.at[pl.ds(nbase, GRP)], idx_c.at[nxt],
                                isem).wait()
        return carry

      lax.fori_loop(0, NGRP, grp, 0)
      plsc.subcore_barrier()
      for t in range(RPT // CHUNK):
        pltpu.sync_copy(acc.at[pl.ds(r0 + t * CHUNK, CHUNK)], g0)
        pltpu.sync_copy(g0, out.at[c, pl.ds(r0 + t * CHUNK, CHUNK)])

  return pl.kernel(body, out_type=out_type, mesh=mesh, scratch_types=scratch)


BLK = 400
NBLK = N // BLK


def _dense_bn_body(s_ref, deg_ref, h_ref, w_ref, r_ref, b_ref, g_ref, be_ref,
                   o_ref, u_ref, stat_ref):
  p = pl.program_id(0)
  i = pl.program_id(1)

  @pl.when(p == 0)
  def _compute():
    @pl.when(i == 0)
    def _init():
      stat_ref[...] = jnp.zeros_like(stat_ref)

    ssum = s_ref[0] + s_ref[1]
    deg = jnp.maximum(deg_ref[0] + deg_ref[1], 1.0)
    agg = ssum / deg
    u = (jnp.dot(agg, w_ref[...], preferred_element_type=jnp.float32)
         + jnp.dot(h_ref[...], r_ref[...], preferred_element_type=jnp.float32)
         + b_ref[...])
    u_ref[pl.ds(i * BLK, BLK), :] = u
    stat_ref[0:1, :] += jnp.sum(u, axis=0, keepdims=True)
    stat_ref[1:2, :] += jnp.sum(u * u, axis=0, keepdims=True)
    o_ref[...] = u

  @pl.when(p == 1)
  def _normalize():
    mu = stat_ref[0:1, :] * (1.0 / N)
    var = stat_ref[1:2, :] * (1.0 / N) - mu * mu
    inv = lax.rsqrt(var + 1e-5)
    u = u_ref[pl.ds(i * BLK, BLK), :]
    o_ref[...] = jnp.maximum((u - mu) * inv * g_ref[...] + be_ref[...], 0.0)


def _dense_bn(s2, deg, h, W, R, b, g, be):
  return pl.pallas_call(
      _dense_bn_body,
      grid=(2, NBLK),
      in_specs=[
          pl.BlockSpec((2, BLK, 128), lambda p, i: (0, i, 0)),
          pl.BlockSpec((2, BLK, 1), lambda p, i: (0, i, 0)),
          pl.BlockSpec((BLK, 128), lambda p, i: (i, 0)),
          pl.BlockSpec((128, 128), lambda p, i: (0, 0)),
          pl.BlockSpec((128, 128), lambda p, i: (0, 0)),
          pl.BlockSpec((1, 128), lambda p, i: (0, 0)),
          pl.BlockSpec((1, 128), lambda p, i: (0, 0)),
          pl.BlockSpec((1, 128), lambda p, i: (0, 0)),
      ],
      out_specs=pl.BlockSpec((BLK, 128), lambda p, i: (i, 0)),
      out_shape=jax.ShapeDtypeStruct((N, 128), jnp.float32),
      scratch_shapes=[
          pltpu.VMEM((N, 128), jnp.float32),
          pltpu.VMEM((2, 128), jnp.float32),
      ],
  )(s2, deg, h, W, R, b, g, be)


def _final_body(s_ref, deg_ref, h_ref, w3_ref, r3_ref, b3_ref, o_ref):
  ssum = s_ref[0] + s_ref[1]
  deg = jnp.maximum(deg_ref[0] + deg_ref[1], 1.0)
  agg = ssum / deg
  u = (jnp.dot(agg, w3_ref[...], preferred_element_type=jnp.float32)
       + jnp.dot(h_ref[...], r3_ref[...], preferred_element_type=jnp.float32)
       + b3_ref[...])
  lane = lax.broadcasted_iota(jnp.int32, u.shape, 1)
  um = jnp.where(lane < 40, u, -1e30)
  m = jnp.max(um, axis=1, keepdims=True)
  lse = jnp.log(jnp.sum(jnp.exp(um - m), axis=1, keepdims=True)) + m
  o_ref[...] = (um - lse)[:, :40]


def _final(s3, deg, h2, W3p, R3p, b3p):
  return pl.pallas_call(
      _final_body,
      grid=(NBLK,),
      in_specs=[
          pl.BlockSpec((2, BLK, 128), lambda i: (0, i, 0)),
          pl.BlockSpec((2, BLK, 1), lambda i: (0, i, 0)),
          pl.BlockSpec((BLK, 128), lambda i: (i, 0)),
          pl.BlockSpec((128, 64), lambda i: (0, 0)),
          pl.BlockSpec((128, 64), lambda i: (0, 0)),
          pl.BlockSpec((1, 64), lambda i: (0, 0)),
      ],
      out_specs=pl.BlockSpec((BLK, 40), lambda i: (i, 0)),
      out_shape=jax.ShapeDtypeStruct((N, 40), jnp.float32),
  )(s3, deg, h2, W3p, R3p, b3p)


_agg128 = _make_agg(128, False)


def kernel(x, edge_index, W1, R1, b1, g1, be1, W2, R2, b2, g2, be2, W3, R3, b3):
  E = edge_index.shape[1]
  row = edge_index[0]
  col = edge_index[1]
  row2d = jnp.concatenate(
      [row, jnp.full((EPAD - E,), N, jnp.int32)]).reshape(NW * CPW, CHUNK)
  col2d = jnp.concatenate(
      [col, jnp.zeros((EPAD - E,), jnp.int32)]).reshape(NW * CPW, CHUNK)
  b1r = b1.reshape(1, 128)
  g1r = g1.reshape(1, 128)
  be1r = be1.reshape(1, 128)
  b2r = b2.reshape(1, 128)
  g2r = g2.reshape(1, 128)
  be2r = be2.reshape(1, 128)
  W3p = jnp.pad(W3, ((0, 0), (0, 24)))
  R3p = jnp.pad(R3, ((0, 0), (0, 24)))
  b3p = jnp.pad(b3, (0, 24)).reshape(1, 64)

  dg = jax.ops.segment_sum(jnp.ones((E,), jnp.float32), row, num_segments=NPAD)
  deg = jnp.stack([dg[:, None], jnp.zeros((NPAD, 1), jnp.float32)])
  (s1,) = _agg128(x, row2d, col2d)
  h1 = _dense_bn(s1, deg, x, W1, R1, b1r, g1r, be1r)
  (s2,) = _agg128(h1, row2d, col2d)
  h2 = _dense_bn(s2, deg, h1, W2, R2, b2r, g2r, be2r)
  (s3,) = _agg128(h2, row2d, col2d)
  return _final(s3, deg, h2, W3p, R3p, b3p)
